# Initial kernel scaffold; baseline (speedup 1.0000x reference)
import functools

import jax
import jax.numpy as jnp
from jax import lax
from jax.experimental import pallas as pl
from jax.experimental.pallas import tpu as pltpu
from jax.experimental.pallas import tpu_sc as plsc

_NC = 2
_NS = 16
_NW = _NC * _NS

_CHUNK = 80


def _sc_segment_sum(x, zeros, src3, dst3):
    n, d = x.shape
    nw, nchunks, chunk = src3.shape
    rows_per_tile = n // _NS
    init_chunk = 125
    init_steps = rows_per_tile // init_chunk

    mesh = plsc.VectorSubcoreMesh(core_axis_name="c", subcore_axis_name="s")

    @functools.partial(
        pl.kernel,
        mesh=mesh,
        out_type=jax.ShapeDtypeStruct((_NC, n, d), jnp.float32),
        scratch_types=[
            pltpu.VMEM((nchunks, chunk), jnp.int32),
            pltpu.VMEM((nchunks, chunk), jnp.int32),
            pltpu.VMEM((chunk, d), jnp.float32),
            pltpu.VMEM((init_chunk, d), jnp.float32),
            pltpu.VMEM_SHARED((n, d), jnp.float32),
            pltpu.SemaphoreType.DMA,
        ],
    )
    def k(x_hbm, z_hbm, src_hbm, dst_hbm, out_hbm,
          src_v, dst_v, rows_v, stage_v, agg_sh, sem):
        c = lax.axis_index("c")
        s = lax.axis_index("s")
        wid = c * _NS + s
        base_row = s * rows_per_tile

        def init_from(ref):
            def body(i, carry):
                r0 = base_row + i * init_chunk
                pltpu.sync_copy(ref.at[pl.ds(r0, init_chunk)], stage_v)
                pltpu.sync_copy(stage_v, agg_sh.at[pl.ds(r0, init_chunk)])
                return carry
            lax.fori_loop(0, init_steps, body, 0)

        @pl.when(c == 0)
        def _():
            init_from(x_hbm)

        @pl.when(c != 0)
        def _():
            init_from(z_hbm)

        plsc.subcore_barrier()

        pltpu.sync_copy(src_hbm.at[wid], src_v)
        pltpu.sync_copy(dst_hbm.at[wid], dst_v)

        def edge_body(j, carry):
            pltpu.async_copy(x_hbm.at[src_v.at[j]], rows_v, sem).wait()
            pltpu.sync_copy(rows_v, agg_sh.at[dst_v.at[j]], add=True)
            return carry
        lax.fori_loop(0, nchunks, edge_body, 0)

        plsc.subcore_barrier()

        def out_body(i, carry):
            r0 = base_row + i * init_chunk
            pltpu.sync_copy(agg_sh.at[pl.ds(r0, init_chunk)], stage_v)
            pltpu.sync_copy(stage_v, out_hbm.at[c, pl.ds(r0, init_chunk)])
            return carry
        lax.fori_loop(0, init_steps, out_body, 0)

    return k(x, zeros, src3, dst3)


def _mlp1(agg, w1, b1):
    _, n, d = agg.shape
    dh = w1.shape[1]
    blk = 1000
    grid = n // blk

    def body(agg_ref, w1_ref, b1_ref, h1_ref, stats_ref, acc_ref):
        i = pl.program_id(0)
        h = agg_ref[0] + agg_ref[1]
        h1 = jnp.dot(h, w1_ref[...], preferred_element_type=jnp.float32)
        h1 = h1 + b1_ref[...]
        h1_ref[...] = h1
        part = jnp.concatenate(
            [jnp.sum(h1, axis=0, keepdims=True),
             jnp.sum(h1 * h1, axis=0, keepdims=True)], axis=0)

        @pl.when(i == 0)
        def _():
            acc_ref[...] = part

        @pl.when(i > 0)
        def _():
            acc_ref[...] += part

        @pl.when(i == pl.num_programs(0) - 1)
        def _():
            stats_ref[...] = acc_ref[...]

    return pl.pallas_call(
        body,
        grid=(grid,),
        in_specs=[
            pl.BlockSpec((2, blk, d), lambda i: (0, i, 0)),
            pl.BlockSpec((d, dh), lambda i: (0, 0)),
            pl.BlockSpec((1, dh), lambda i: (0, 0)),
        ],
        out_specs=[
            pl.BlockSpec((blk, dh), lambda i: (i, 0)),
            pl.BlockSpec((2, dh), lambda i: (0, 0)),
        ],
        out_shape=[
            jax.ShapeDtypeStruct((n, dh), jnp.float32),
            jax.ShapeDtypeStruct((2, dh), jnp.float32),
        ],
        scratch_shapes=[pltpu.VMEM((2, dh), jnp.float32)],
    )(agg, w1, b1)


def _mlp2(h1, stats, n_rows, gamma, beta, w2, b2):
    n, dh = h1.shape
    do = w2.shape[1]
    blk = 1000
    grid = n // blk
    inv_n = 1.0 / n_rows

    def body(h1_ref, stats_ref, g_ref, be_ref, w2_ref, b2_ref, out_ref):
        mean = stats_ref[0:1, :] * inv_n
        ex2 = stats_ref[1:2, :] * inv_n
        var = ex2 - mean * mean
        scale = lax.rsqrt(var + 1e-5) * g_ref[...]
        h = (h1_ref[...] - mean) * scale + be_ref[...]
        h = jnp.maximum(h, 0.0)
        out_ref[...] = jnp.dot(h, w2_ref[...],
                               preferred_element_type=jnp.float32) + b2_ref[...]

    return pl.pallas_call(
        body,
        grid=(grid,),
        in_specs=[
            pl.BlockSpec((blk, dh), lambda i: (i, 0)),
            pl.BlockSpec((2, dh), lambda i: (0, 0)),
            pl.BlockSpec((1, dh), lambda i: (0, 0)),
            pl.BlockSpec((1, dh), lambda i: (0, 0)),
            pl.BlockSpec((dh, do), lambda i: (0, 0)),
            pl.BlockSpec((1, do), lambda i: (0, 0)),
        ],
        out_specs=pl.BlockSpec((blk, do), lambda i: (i, 0)),
        out_shape=jax.ShapeDtypeStruct((n, do), jnp.float32),
    )(h1, stats, gamma, beta, w2, b2)


def kernel(x, edge_index, W1, b1, gamma, beta, W2, b2):
    n, d = x.shape
    e = edge_index.shape[1]
    per_worker = e // _NW
    nchunks = per_worker // _CHUNK

    src3 = edge_index[0].reshape(_NW, nchunks, _CHUNK)
    dst3 = edge_index[1].reshape(_NW, nchunks, _CHUNK)
    zeros = jnp.zeros_like(x)

    agg = _sc_segment_sum(x, zeros, src3, dst3)
    h1, stats = _mlp1(agg, W1, b1.reshape(1, -1))
    out = _mlp2(h1, stats, n, gamma.reshape(1, -1), beta.reshape(1, -1),
                W2, b2.reshape(1, -1))
    return out


# speedup vs baseline: 3.3248x; 3.3248x over previous
import functools

import jax
import jax.numpy as jnp
from jax import lax
from jax.experimental import pallas as pl
from jax.experimental.pallas import tpu as pltpu
from jax.experimental.pallas import tpu_sc as plsc

_NC = 2
_NS = 16
_NW = _NC * _NS

_CHUNK = 128
_GROUP = 16


def _sc_segment_sum(x_pad, zeros_pad, src3, dst3):
    n_p, d = x_pad.shape
    nw, nchunks, chunk = src3.shape
    ngroups = nchunks // _GROUP
    rows_per_tile = n_p // _NS
    init_chunk = 128
    init_steps = rows_per_tile // init_chunk

    mesh = plsc.VectorSubcoreMesh(core_axis_name="c", subcore_axis_name="s")

    @functools.partial(
        pl.kernel,
        mesh=mesh,
        out_type=jax.ShapeDtypeStruct((_NC, n_p, d), jnp.float32),
        scratch_types=[
            pltpu.VMEM((_GROUP, chunk), jnp.int32),
            pltpu.VMEM((_GROUP, chunk), jnp.int32),
            pltpu.VMEM((chunk, d), jnp.float32),
            pltpu.VMEM_SHARED((n_p, d), jnp.float32),
            pltpu.SemaphoreType.DMA,
        ],
    )
    def k(x_hbm, z_hbm, src_hbm, dst_hbm, out_hbm,
          src_v, dst_v, rows_v, agg_sh, sem):
        c = lax.axis_index("c")
        s = lax.axis_index("s")
        wid = c * _NS + s
        base_row = s * rows_per_tile

        def init_from(ref):
            def body(i, carry):
                r0 = base_row + i * init_chunk
                pltpu.sync_copy(ref.at[pl.ds(r0, init_chunk)], rows_v)
                pltpu.sync_copy(rows_v, agg_sh.at[pl.ds(r0, init_chunk)])
                return carry
            lax.fori_loop(0, init_steps, body, 0)

        @pl.when(c == 0)
        def _():
            init_from(x_hbm)

        @pl.when(c != 0)
        def _():
            init_from(z_hbm)

        plsc.subcore_barrier()

        def group_body(g, carry):
            pltpu.sync_copy(src_hbm.at[wid, pl.ds(g * _GROUP, _GROUP)], src_v)
            pltpu.sync_copy(dst_hbm.at[wid, pl.ds(g * _GROUP, _GROUP)], dst_v)

            def edge_body(j, carry2):
                pltpu.async_copy(x_hbm.at[src_v.at[j]], rows_v, sem).wait()
                pltpu.sync_copy(rows_v, agg_sh.at[dst_v.at[j]], add=True)
                return carry2
            lax.fori_loop(0, _GROUP, edge_body, 0)
            return carry
        lax.fori_loop(0, ngroups, group_body, 0)

        plsc.subcore_barrier()

        def out_body(i, carry):
            r0 = base_row + i * init_chunk
            pltpu.sync_copy(agg_sh.at[pl.ds(r0, init_chunk)], rows_v)
            pltpu.sync_copy(rows_v, out_hbm.at[c, pl.ds(r0, init_chunk)])
            return carry
        lax.fori_loop(0, init_steps, out_body, 0)

    return k(x_pad, zeros_pad, src3, dst3)


def _mlp1(agg, w1, b1, n_valid):
    _, n_p, d = agg.shape
    dh = w1.shape[1]
    blk = 1024
    grid = n_p // blk

    def body(agg_ref, w1_ref, b1_ref, h1_ref, stats_ref, acc_ref):
        i = pl.program_id(0)
        h = agg_ref[0] + agg_ref[1]
        h1 = jnp.dot(h, w1_ref[...], preferred_element_type=jnp.float32)
        h1 = h1 + b1_ref[...]
        h1_ref[...] = h1
        rows = lax.broadcasted_iota(jnp.int32, (blk, 1), 0) + i * blk
        h1m = jnp.where(rows < n_valid, h1, 0.0)
        part = jnp.concatenate(
            [jnp.sum(h1m, axis=0, keepdims=True),
             jnp.sum(h1m * h1m, axis=0, keepdims=True)], axis=0)

        @pl.when(i == 0)
        def _():
            acc_ref[...] = part

        @pl.when(i > 0)
        def _():
            acc_ref[...] += part

        @pl.when(i == pl.num_programs(0) - 1)
        def _():
            stats_ref[...] = acc_ref[...]

    return pl.pallas_call(
        body,
        grid=(grid,),
        in_specs=[
            pl.BlockSpec((2, blk, d), lambda i: (0, i, 0)),
            pl.BlockSpec((d, dh), lambda i: (0, 0)),
            pl.BlockSpec((1, dh), lambda i: (0, 0)),
        ],
        out_specs=[
            pl.BlockSpec((blk, dh), lambda i: (i, 0)),
            pl.BlockSpec((2, dh), lambda i: (0, 0)),
        ],
        out_shape=[
            jax.ShapeDtypeStruct((n_p, dh), jnp.float32),
            jax.ShapeDtypeStruct((2, dh), jnp.float32),
        ],
        scratch_shapes=[pltpu.VMEM((2, dh), jnp.float32)],
    )(agg, w1, b1)


def _mlp2(h1, stats, n_valid, gamma, beta, w2, b2):
    n_p, dh = h1.shape
    do = w2.shape[1]
    blk = 1024
    grid = n_p // blk
    inv_n = 1.0 / n_valid

    def body(h1_ref, stats_ref, g_ref, be_ref, w2_ref, b2_ref, out_ref):
        mean = stats_ref[0:1, :] * inv_n
        ex2 = stats_ref[1:2, :] * inv_n
        var = ex2 - mean * mean
        scale = lax.rsqrt(var + 1e-5) * g_ref[...]
        h = (h1_ref[...] - mean) * scale + be_ref[...]
        h = jnp.maximum(h, 0.0)
        out_ref[...] = jnp.dot(h, w2_ref[...],
                               preferred_element_type=jnp.float32) + b2_ref[...]

    return pl.pallas_call(
        body,
        grid=(grid,),
        in_specs=[
            pl.BlockSpec((blk, dh), lambda i: (i, 0)),
            pl.BlockSpec((2, dh), lambda i: (0, 0)),
            pl.BlockSpec((1, dh), lambda i: (0, 0)),
            pl.BlockSpec((1, dh), lambda i: (0, 0)),
            pl.BlockSpec((dh, do), lambda i: (0, 0)),
            pl.BlockSpec((1, do), lambda i: (0, 0)),
        ],
        out_specs=pl.BlockSpec((blk, do), lambda i: (i, 0)),
        out_shape=jax.ShapeDtypeStruct((n_p, do), jnp.float32),
    )(h1, stats, gamma, beta, w2, b2)


def kernel(x, edge_index, W1, b1, gamma, beta, W2, b2):
    n, d = x.shape
    e = edge_index.shape[1]
    per_worker = e // _NW
    per_worker_p = -(-per_worker // (_CHUNK * _GROUP)) * (_CHUNK * _GROUP)
    nchunks = per_worker_p // _CHUNK
    pad_e = per_worker_p - per_worker

    n_p = -(-(n + 1) // (_NS * 128)) * (_NS * 128)
    x_pad = jnp.pad(x, ((0, n_p - n), (0, 0)))
    zeros_pad = jnp.zeros((n_p, d), jnp.float32)

    src2 = edge_index[0].reshape(_NW, per_worker)
    dst2 = edge_index[1].reshape(_NW, per_worker)
    src3 = jnp.pad(src2, ((0, 0), (0, pad_e))).reshape(_NW, nchunks, _CHUNK)
    dst3 = jnp.pad(dst2, ((0, 0), (0, pad_e)),
                   constant_values=n).reshape(_NW, nchunks, _CHUNK)

    agg = _sc_segment_sum(x_pad, zeros_pad, src3, dst3)
    h1, stats = _mlp1(agg, W1, b1.reshape(1, -1), n)
    out = _mlp2(h1, stats, n, gamma.reshape(1, -1), beta.reshape(1, -1),
                W2, b2.reshape(1, -1))
    return out[:n]


# speedup vs baseline: 3.5542x; 1.0690x over previous
import functools

import jax
import jax.numpy as jnp
from jax import lax
from jax.experimental import pallas as pl
from jax.experimental.pallas import tpu as pltpu
from jax.experimental.pallas import tpu_sc as plsc

_NC = 2
_NS = 16
_NW = _NC * _NS

_CHUNK = 128
_GROUP = 8


def _sc_segment_sum(x_pad, zeros_pad, src3, dst3):
    n_p, d = x_pad.shape
    nw, nchunks, chunk = src3.shape
    ngroups = nchunks // _GROUP
    rows_per_tile = n_p // _NS
    init_chunk = 128
    init_steps = rows_per_tile // init_chunk

    mesh = plsc.VectorSubcoreMesh(core_axis_name="c", subcore_axis_name="s")

    @functools.partial(
        pl.kernel,
        mesh=mesh,
        out_type=jax.ShapeDtypeStruct((_NC, n_p, d), jnp.float32),
        scratch_types=[
            pltpu.VMEM((_GROUP, chunk), jnp.int32),
            pltpu.VMEM((_GROUP, chunk), jnp.int32),
            pltpu.VMEM((chunk, d), jnp.float32),
            pltpu.VMEM((chunk, d), jnp.float32),
            pltpu.VMEM_SHARED((n_p, d), jnp.float32),
            pltpu.SemaphoreType.DMA,
        ],
    )
    def k(x_hbm, z_hbm, src_hbm, dst_hbm, out_hbm,
          src_v, dst_v, rows_v, rows_w, agg_sh, sem):
        c = lax.axis_index("c")
        s = lax.axis_index("s")
        wid = c * _NS + s
        base_row = s * rows_per_tile

        def init_from(ref):
            def body(i, carry):
                r0 = base_row + i * init_chunk
                pltpu.sync_copy(ref.at[pl.ds(r0, init_chunk)], rows_v)
                pltpu.sync_copy(rows_v, agg_sh.at[pl.ds(r0, init_chunk)])
                return carry
            lax.fori_loop(0, init_steps, body, 0)

        @pl.when(c == 0)
        def _():
            init_from(x_hbm)

        @pl.when(c != 0)
        def _():
            init_from(z_hbm)

        plsc.subcore_barrier()

        def group_body(g, carry):
            pltpu.sync_copy(src_hbm.at[wid, pl.ds(g * _GROUP, _GROUP)], src_v)
            pltpu.sync_copy(dst_hbm.at[wid, pl.ds(g * _GROUP, _GROUP)], dst_v)

            bufs = [rows_v, rows_w]
            cp = pltpu.async_copy(x_hbm.at[src_v.at[0]], bufs[0], sem)
            for j in range(1, _GROUP):
                cp.wait()
                nxt = pltpu.async_copy(x_hbm.at[src_v.at[j]], bufs[j % 2], sem)
                pltpu.sync_copy(bufs[(j - 1) % 2],
                                agg_sh.at[dst_v.at[j - 1]], add=True)
                cp = nxt
            cp.wait()
            pltpu.sync_copy(bufs[(_GROUP - 1) % 2],
                            agg_sh.at[dst_v.at[_GROUP - 1]], add=True)
            return carry
        lax.fori_loop(0, ngroups, group_body, 0)

        plsc.subcore_barrier()

        def out_body(i, carry):
            r0 = base_row + i * init_chunk
            pltpu.sync_copy(agg_sh.at[pl.ds(r0, init_chunk)], rows_v)
            pltpu.sync_copy(rows_v, out_hbm.at[c, pl.ds(r0, init_chunk)])
            return carry
        lax.fori_loop(0, init_steps, out_body, 0)

    return k(x_pad, zeros_pad, src3, dst3)


def _mlp1(agg, w1, b1, n_valid):
    _, n_p, d = agg.shape
    dh = w1.shape[1]
    blk = 1024
    grid = n_p // blk

    def body(agg_ref, w1_ref, b1_ref, h1_ref, stats_ref, acc_ref):
        i = pl.program_id(0)
        h = agg_ref[0] + agg_ref[1]
        h1 = jnp.dot(h, w1_ref[...], preferred_element_type=jnp.float32)
        h1 = h1 + b1_ref[...]
        h1_ref[...] = h1
        rows = lax.broadcasted_iota(jnp.int32, (blk, 1), 0) + i * blk
        h1m = jnp.where(rows < n_valid, h1, 0.0)
        part = jnp.concatenate(
            [jnp.sum(h1m, axis=0, keepdims=True),
             jnp.sum(h1m * h1m, axis=0, keepdims=True)], axis=0)

        @pl.when(i == 0)
        def _():
            acc_ref[...] = part

        @pl.when(i > 0)
        def _():
            acc_ref[...] += part

        @pl.when(i == pl.num_programs(0) - 1)
        def _():
            stats_ref[...] = acc_ref[...]

    return pl.pallas_call(
        body,
        grid=(grid,),
        in_specs=[
            pl.BlockSpec((2, blk, d), lambda i: (0, i, 0)),
            pl.BlockSpec((d, dh), lambda i: (0, 0)),
            pl.BlockSpec((1, dh), lambda i: (0, 0)),
        ],
        out_specs=[
            pl.BlockSpec((blk, dh), lambda i: (i, 0)),
            pl.BlockSpec((2, dh), lambda i: (0, 0)),
        ],
        out_shape=[
            jax.ShapeDtypeStruct((n_p, dh), jnp.float32),
            jax.ShapeDtypeStruct((2, dh), jnp.float32),
        ],
        scratch_shapes=[pltpu.VMEM((2, dh), jnp.float32)],
    )(agg, w1, b1)


def _mlp2(h1, stats, n_valid, gamma, beta, w2, b2):
    n_p, dh = h1.shape
    do = w2.shape[1]
    blk = 1024
    grid = n_p // blk
    inv_n = 1.0 / n_valid

    def body(h1_ref, stats_ref, g_ref, be_ref, w2_ref, b2_ref, out_ref):
        mean = stats_ref[0:1, :] * inv_n
        ex2 = stats_ref[1:2, :] * inv_n
        var = ex2 - mean * mean
        scale = lax.rsqrt(var + 1e-5) * g_ref[...]
        h = (h1_ref[...] - mean) * scale + be_ref[...]
        h = jnp.maximum(h, 0.0)
        out_ref[...] = jnp.dot(h, w2_ref[...],
                               preferred_element_type=jnp.float32) + b2_ref[...]

    return pl.pallas_call(
        body,
        grid=(grid,),
        in_specs=[
            pl.BlockSpec((blk, dh), lambda i: (i, 0)),
            pl.BlockSpec((2, dh), lambda i: (0, 0)),
            pl.BlockSpec((1, dh), lambda i: (0, 0)),
            pl.BlockSpec((1, dh), lambda i: (0, 0)),
            pl.BlockSpec((dh, do), lambda i: (0, 0)),
            pl.BlockSpec((1, do), lambda i: (0, 0)),
        ],
        out_specs=pl.BlockSpec((blk, do), lambda i: (i, 0)),
        out_shape=jax.ShapeDtypeStruct((n_p, do), jnp.float32),
    )(h1, stats, gamma, beta, w2, b2)


def kernel(x, edge_index, W1, b1, gamma, beta, W2, b2):
    n, d = x.shape
    e = edge_index.shape[1]
    per_worker = e // _NW
    per_worker_p = -(-per_worker // (_CHUNK * _GROUP)) * (_CHUNK * _GROUP)
    nchunks = per_worker_p // _CHUNK
    pad_e = per_worker_p - per_worker

    n_p = -(-(n + 1) // (_NS * 128)) * (_NS * 128)
    x_pad = jnp.pad(x, ((0, n_p - n), (0, 0)))
    zeros_pad = jnp.zeros((n_p, d), jnp.float32)

    src2 = edge_index[0].reshape(_NW, per_worker)
    dst2 = edge_index[1].reshape(_NW, per_worker)
    src3 = jnp.pad(src2, ((0, 0), (0, pad_e))).reshape(_NW, nchunks, _CHUNK)
    dst3 = jnp.pad(dst2, ((0, 0), (0, pad_e)),
                   constant_values=n).reshape(_NW, nchunks, _CHUNK)

    agg = _sc_segment_sum(x_pad, zeros_pad, src3, dst3)
    h1, stats = _mlp1(agg, W1, b1.reshape(1, -1), n)
    out = _mlp2(h1, stats, n, gamma.reshape(1, -1), beta.reshape(1, -1),
                W2, b2.reshape(1, -1))
    return out[:n]
